# baseline (speedup 1.0000x reference)
import functools

import jax
import jax.numpy as jnp
from jax import lax
from jax.experimental import pallas as pl
from jax.experimental.pallas import tpu as pltpu
from jax.experimental.pallas import tpu_sc as plsc

B_ROWS = 4096
SEQ = 200
DIM = 64
NUM_IDX = B_ROWS * SEQ

NC = 2
NS = 16
NW = NC * NS

PER_W = NUM_IDX // NW
CHUNK = 1024
N_CHUNKS = PER_W // CHUNK

_mesh = plsc.VectorSubcoreMesh(core_axis_name="c", subcore_axis_name="s")


@functools.partial(
    pl.kernel,
    out_type=jax.ShapeDtypeStruct((NUM_IDX, DIM), jnp.float32),
    mesh=_mesh,
    scratch_types=[
        pltpu.VMEM((CHUNK,), jnp.int32),
        pltpu.VMEM((CHUNK, DIM), jnp.float32),
        pltpu.SemaphoreType.DMA,
    ],
    compiler_params=pltpu.CompilerParams(use_tc_tiling_on_sc=False),
)
def _gather_kernel(idx_hbm, table_hbm, out_hbm, idx_v, rows_v, sem):
    wid = lax.axis_index("s") * NC + lax.axis_index("c")
    base = wid * PER_W

    def body(i, _):
        off = base + i * CHUNK
        pltpu.sync_copy(idx_hbm.at[pl.ds(off, CHUNK)], idx_v)
        pltpu.async_copy(table_hbm.at[idx_v], rows_v, sem).wait()
        pltpu.sync_copy(rows_v, out_hbm.at[pl.ds(off, CHUNK)])
        return ()

    lax.fori_loop(0, N_CHUNKS, body, (), unroll=False)


def kernel(x, weight):
    idx = x.reshape(-1).astype(jnp.int32)
    out = _gather_kernel(idx, weight)
    return out.reshape(B_ROWS, SEQ, DIM)


# speedup vs baseline: 1.0170x; 1.0170x over previous
import functools

import jax
import jax.numpy as jnp
from jax import lax
from jax.experimental import pallas as pl
from jax.experimental.pallas import tpu as pltpu
from jax.experimental.pallas import tpu_sc as plsc

B_ROWS = 4096
SEQ = 200
DIM = 64
NUM_IDX = B_ROWS * SEQ

NC = 2
NS = 16
NW = NC * NS

PER_W = NUM_IDX // NW
CHUNK = 256
NBUF = 4
DELAY = 2
N_CHUNKS = PER_W // CHUNK
N_GROUPS = N_CHUNKS // NBUF

_mesh = plsc.VectorSubcoreMesh(core_axis_name="c", subcore_axis_name="s")


@functools.partial(
    pl.kernel,
    out_type=jax.ShapeDtypeStruct((NUM_IDX, DIM), jnp.float32),
    mesh=_mesh,
    scratch_types=[
        pltpu.VMEM((PER_W,), jnp.int32),
        [pltpu.VMEM((CHUNK, DIM), jnp.float32) for _ in range(NBUF)],
        [pltpu.SemaphoreType.DMA for _ in range(NBUF)],
        [pltpu.SemaphoreType.DMA for _ in range(NBUF)],
    ],
    compiler_params=pltpu.CompilerParams(use_tc_tiling_on_sc=False),
)
def _gather_kernel(idx_hbm, table_hbm, out_hbm, idx_v, rows, sem_g, sem_s):
    wid = lax.axis_index("s") * NC + lax.axis_index("c")
    base = wid * PER_W

    pltpu.sync_copy(idx_hbm.at[pl.ds(base, PER_W)], idx_v)

    def start_gather(chunk, b):
        off = pl.multiple_of(chunk * CHUNK, CHUNK)
        pltpu.async_copy(
            table_hbm.at[idx_v.at[pl.ds(off, CHUNK)]], rows[b], sem_g[b]
        )

    def wait_gather(b):
        pltpu.make_async_copy(
            table_hbm.at[idx_v.at[pl.ds(0, CHUNK)]], rows[b], sem_g[b]
        ).wait()

    def start_store(chunk, b):
        off = pl.multiple_of(base + chunk * CHUNK, CHUNK)
        pltpu.async_copy(rows[b], out_hbm.at[pl.ds(off, CHUNK)], sem_s[b])

    def wait_store(b):
        pltpu.make_async_copy(
            rows[b], out_hbm.at[pl.ds(base, CHUNK)], sem_s[b]
        ).wait()

    for b in range(NBUF):
        start_gather(b, b)
        if b >= DELAY:
            j = b - DELAY
            wait_gather(j % NBUF)
            start_store(j, j % NBUF)

    def body(g, _):
        for b in range(NBUF):
            i = g * NBUF + b
            wait_store(b)
            start_gather(i, b)
            bj = (b - DELAY) % NBUF
            wait_gather(bj)
            start_store(i - DELAY, bj)
        return ()

    lax.fori_loop(1, N_GROUPS, body, (), unroll=False)

    last = N_CHUNKS - DELAY
    for k in range(DELAY):
        j = last + k
        b = j % NBUF
        wait_gather(b)
        start_store(j, b)
    for b in range(NBUF):
        wait_store(b)


def kernel(x, weight):
    idx = x.reshape(-1).astype(jnp.int32)
    out = _gather_kernel(idx, weight)
    return out.reshape(B_ROWS, SEQ, DIM)


# speedup vs baseline: 1.0171x; 1.0001x over previous
import functools

import jax
import jax.numpy as jnp
from jax import lax
from jax.experimental import pallas as pl
from jax.experimental.pallas import tpu as pltpu
from jax.experimental.pallas import tpu_sc as plsc

B_ROWS = 4096
SEQ = 200
DIM = 64

NC = 2
NS = 16
NW = NC * NS

ROWS_W = B_ROWS // NW
NBUF = 4
DELAY = 2
N_GROUPS = ROWS_W // NBUF

_mesh = plsc.VectorSubcoreMesh(core_axis_name="c", subcore_axis_name="s")


@functools.partial(
    pl.kernel,
    out_type=jax.ShapeDtypeStruct((B_ROWS, SEQ, DIM), jnp.float32),
    mesh=_mesh,
    scratch_types=[
        pltpu.VMEM((ROWS_W, SEQ), jnp.int32),
        [pltpu.VMEM((SEQ, DIM), jnp.float32) for _ in range(NBUF)],
        [pltpu.SemaphoreType.DMA for _ in range(NBUF)],
        [pltpu.SemaphoreType.DMA for _ in range(NBUF)],
    ],
    compiler_params=pltpu.CompilerParams(use_tc_tiling_on_sc=False),
)
def _gather_kernel(idx_hbm, table_hbm, out_hbm, idx_v, rows, sem_g, sem_s):
    wid = lax.axis_index("s") * NC + lax.axis_index("c")
    base = wid * ROWS_W

    pltpu.sync_copy(idx_hbm.at[pl.ds(base, ROWS_W)], idx_v)

    def start_gather(chunk, b):
        pltpu.async_copy(table_hbm.at[idx_v.at[chunk]], rows[b], sem_g[b])

    def wait_gather(b):
        pltpu.make_async_copy(
            table_hbm.at[idx_v.at[0]], rows[b], sem_g[b]
        ).wait()

    def start_store(chunk, b):
        pltpu.async_copy(rows[b], out_hbm.at[base + chunk], sem_s[b])

    def wait_store(b):
        pltpu.make_async_copy(rows[b], out_hbm.at[0], sem_s[b]).wait()

    for b in range(NBUF):
        start_gather(b, b)
        if b >= DELAY:
            j = b - DELAY
            wait_gather(j % NBUF)
            start_store(j, j % NBUF)

    def body(g, _):
        for b in range(NBUF):
            i = g * NBUF + b
            wait_store(b)
            start_gather(i, b)
            bj = (b - DELAY) % NBUF
            wait_gather(bj)
            start_store(i - DELAY, bj)
        return ()

    lax.fori_loop(1, N_GROUPS, body, (), unroll=False)

    last = ROWS_W - DELAY
    for k in range(DELAY):
        j = last + k
        b = j % NBUF
        wait_gather(b)
        start_store(j, b)
    for b in range(NBUF):
        wait_store(b)


def kernel(x, weight):
    return _gather_kernel(x.astype(jnp.int32), weight)
